# baseline (speedup 1.0000x reference)
import functools

import jax
import jax.numpy as jnp
import numpy as np
from jax import lax
from jax.experimental import pallas as pl
from jax.experimental.pallas import tpu as pltpu
from jax.experimental.pallas import tpu_sc as plsc

_LO = float(np.log(0.001 / 0.999))
_HI = -_LO

_NW = 32
_SPW = 16


def _tc_body(cm_ref, cb_ref, ctx_ref, w_ref, lg_ref, p_ref, roff_ref,
             *, sb, m, nb):
    nbit = sb * m
    nrow = sb * nb
    f32 = jnp.float32

    cm = cm_ref[...].reshape(nbit, cm_ref.shape[-1])
    d = jnp.dot(cm, ctx_ref[...], preferred_element_type=f32)
    cb = cb_ref[...].reshape(nbit, 1)
    bits = (d > cb).astype(f32)
    r = lax.broadcasted_iota(jnp.int32, (sb, nbit), 1)
    s = lax.broadcasted_iota(jnp.int32, (sb, nbit), 0)
    a4 = jnp.where(r // m == s, lax.shift_left(1, r % m).astype(f32), 0.0)
    idx = jnp.dot(a4, bits, preferred_element_type=f32)

    srow = lax.broadcasted_iota(jnp.int32, idx.shape, 0) % _SPW
    bcol = lax.broadcasted_iota(jnp.int32, idx.shape, 1)
    roff_ref[...] = (srow * nb + idx.astype(jnp.int32)) * idx.shape[1] + bcol

    w = w_ref[...].reshape(nrow, w_ref.shape[-1])
    p_ref[...] = jnp.dot(w, lg_ref[...], preferred_element_type=f32)


def _sc_select(p_hbm, roff_hbm, out_hbm, chunk, offs, outbuf):
    wid = lax.axis_index("s") * 2 + lax.axis_index("c")
    nel = _SPW * 16 * 64
    pltpu.sync_copy(p_hbm.at[pl.ds(wid * nel, nel)], chunk)
    pltpu.sync_copy(roff_hbm.at[pl.ds(wid * _SPW, _SPW)], offs)
    for sl in range(_SPW):
        for j in range(4):
            off = offs[sl, pl.ds(16 * j, 16)]
            val = plsc.load_gather(chunk, [off])
            val = jnp.minimum(jnp.maximum(val, _LO), _HI)
            outbuf[sl, pl.ds(16 * j, 16)] = val
    pltpu.sync_copy(outbuf, out_hbm.at[pl.ds(wid * _SPW, _SPW)])


def kernel(logit, context, context_maps, context_bias, weights, bias,
           boolean_converter):
    B, I = logit.shape
    _, C = context.shape
    K, S, M, _ = context_maps.shape
    NB = weights.shape[2]
    N = K * S
    SB = 32
    G = -(-N // SB)
    NP = G * SB

    cm = context_maps.reshape(N, M, C)
    cb = context_bias.reshape(N, M, 1)
    wt = weights.reshape(N, NB, I)
    ctxT = context.T
    lgT = logit.T

    p, roff = pl.pallas_call(
        functools.partial(_tc_body, sb=SB, m=M, nb=NB),
        grid=(G,),
        in_specs=[
            pl.BlockSpec((SB, M, C), lambda i: (i, 0, 0)),
            pl.BlockSpec((SB, M, 1), lambda i: (i, 0, 0)),
            pl.BlockSpec((C, B), lambda i: (0, 0)),
            pl.BlockSpec((SB, NB, I), lambda i: (i, 0, 0)),
            pl.BlockSpec((I, B), lambda i: (0, 0)),
        ],
        out_specs=[
            pl.BlockSpec((SB * NB, B), lambda i: (i, 0)),
            pl.BlockSpec((SB, B), lambda i: (i, 0)),
        ],
        out_shape=[
            jax.ShapeDtypeStruct((NP * NB, B), jnp.float32),
            jax.ShapeDtypeStruct((NP, B), jnp.int32),
        ],
    )(cm, cb, ctxT, wt, lgT)

    p2 = p.reshape(NP * NB * B)
    sel = pl.kernel(
        _sc_select,
        out_type=jax.ShapeDtypeStruct((NP, B), jnp.float32),
        mesh=plsc.VectorSubcoreMesh(core_axis_name="c", subcore_axis_name="s"),
        compiler_params=pltpu.CompilerParams(needs_layout_passes=False),
        scratch_types=[
            pltpu.VMEM((_SPW * 16 * B,), jnp.float32),
            pltpu.VMEM((_SPW, B), jnp.int32),
            pltpu.VMEM((_SPW, B), jnp.float32),
        ],
    )(p2, roff)

    body = sel[:N].reshape(K, S, B).transpose(2, 1, 0)
    bias_append = jnp.broadcast_to(bias, (B, 1, K))
    return jnp.concatenate([bias_append, body], axis=1)
